# baseline (speedup 1.0000x reference)
import functools

import jax
import jax.numpy as jnp
from jax import lax
from jax.experimental import pallas as pl
from jax.experimental.pallas import tpu as pltpu
from jax.experimental.pallas import tpu_sc as plsc

_HIDDEN = 1024
_LANES = 16
_NCORES = 2
_NSUB = 16
_NW = _NCORES * _NSUB

_CHUNK = 16


def _emb_body(ids_hbm, pids_hbm, word_hbm, ptab_hbm, out_hbm,
              idx_w, idx_p, w0, w1, p0, p1, o0, o1,
              sw0, sw1, sp0, sp1, so0, so1, *, per_w, nchunk, wpb):
    w = (w0, w1)
    p = (p0, p1)
    o = (o0, o1)
    sw = (sw0, sw1)
    sp = (sp0, sp1)
    so = (so0, so1)

    wid = lax.axis_index("s") * _NCORES + lax.axis_index("c")
    bi = wid // wpb
    co = (wid % wpb) * per_w
    pltpu.sync_copy(ids_hbm.at[bi, pl.ds(co, per_w)], idx_w)
    pltpu.sync_copy(pids_hbm.at[bi, pl.ds(co, per_w)], idx_p)

    def gather_pair(ci, b):
        off = ci * _CHUNK
        pltpu.make_async_copy(word_hbm.at[idx_w.at[pl.ds(off, _CHUNK)]],
                              w[b], sw[b]).start()
        pltpu.make_async_copy(ptab_hbm.at[idx_p.at[pl.ds(off, _CHUNK)]],
                              p[b], sp[b]).start()

    def wait_gather(ci, b):
        off = ci * _CHUNK
        pltpu.make_async_copy(word_hbm.at[idx_w.at[pl.ds(off, _CHUNK)]],
                              w[b], sw[b]).wait()
        pltpu.make_async_copy(ptab_hbm.at[idx_p.at[pl.ds(off, _CHUNK)]],
                              p[b], sp[b]).wait()

    def start_out(ci, b):
        pltpu.make_async_copy(
            o[b], out_hbm.at[bi, pl.ds(co + ci * _CHUNK, _CHUNK)],
            so[b]).start()

    def wait_out(ci, b):
        pltpu.make_async_copy(
            o[b], out_hbm.at[bi, pl.ds(co + ci * _CHUNK, _CHUNK)],
            so[b]).wait()

    gather_pair(0, 0)

    def pair_body(g, carry):
        for b in (0, 1):
            ci = 2 * g + b
            if b == 0:
                gather_pair(ci + 1, 1 - b)
            else:
                @pl.when(g < (nchunk // 2) - 1)
                def _():
                    gather_pair(ci + 1, 1 - b)
            wait_gather(ci, b)

            @pl.when(g > 0)
            def _():
                wait_out(ci - 2, b)

            def add_row(r, c2):
                for j in range(_HIDDEN // 32):
                    v = p[b][r, pl.ds(j * _LANES, _LANES)]
                    pa = plsc.bitcast(jnp.left_shift(v, 16), jnp.float32)
                    pbv = plsc.bitcast(
                        jnp.bitwise_and(v, jnp.int32(-65536)), jnp.float32)
                    sa = pl.ds(j * _LANES, _LANES)
                    sb = pl.ds(_HIDDEN // 2 + j * _LANES, _LANES)
                    o[b][r, sa] = w[b][r, sa] + pa
                    o[b][r, sb] = w[b][r, sb] + pbv
                return c2

            lax.fori_loop(0, _CHUNK, add_row, 0, unroll=False)
            start_out(ci, b)
        return carry

    lax.fori_loop(0, nchunk // 2, pair_body, 0, unroll=False)
    wait_out(nchunk - 2, 0)
    wait_out(nchunk - 1, 1)


def kernel(input_ids, position_ids, word_table, pos_table):
    b, s = input_ids.shape
    n = b * s
    per_w = n // _NW
    nchunk = per_w // _CHUNK
    wpb = s // per_w
    if input_ids.dtype != jnp.int32:
        input_ids = input_ids.astype(jnp.int32)
    if position_ids.dtype != jnp.int32:
        position_ids = position_ids.astype(jnp.int32)

    h2 = _HIDDEN // 2
    pb16 = pos_table.astype(jnp.bfloat16)
    pk = jax.lax.bitcast_convert_type(
        jnp.stack([pb16[:, :h2], pb16[:, h2:]], axis=-1), jnp.int32)

    mesh = plsc.VectorSubcoreMesh(core_axis_name="c", subcore_axis_name="s")
    scratch = [pltpu.VMEM((per_w,), jnp.int32),
               pltpu.VMEM((per_w,), jnp.int32),
               pltpu.VMEM((_CHUNK, _HIDDEN), jnp.float32),
               pltpu.VMEM((_CHUNK, _HIDDEN), jnp.float32),
               pltpu.VMEM((_CHUNK, _HIDDEN // 2), jnp.int32),
               pltpu.VMEM((_CHUNK, _HIDDEN // 2), jnp.int32),
               pltpu.VMEM((_CHUNK, _HIDDEN), jnp.float32),
               pltpu.VMEM((_CHUNK, _HIDDEN), jnp.float32)]
    scratch += [pltpu.SemaphoreType.DMA for _ in range(6)]
    grid_kernel = pl.kernel(
        functools.partial(_emb_body, per_w=per_w, nchunk=nchunk, wpb=wpb),
        mesh=mesh,
        out_type=jax.ShapeDtypeStruct((b, s, _HIDDEN), jnp.float32),
        scratch_types=scratch,
        compiler_params=pltpu.CompilerParams(needs_layout_passes=False),
    )
    return grid_kernel(input_ids, position_ids, word_table, pk)


# speedup vs baseline: 1.2955x; 1.2955x over previous
import functools

import jax
import jax.numpy as jnp
from jax import lax
from jax.experimental import pallas as pl
from jax.experimental.pallas import tpu as pltpu
from jax.experimental.pallas import tpu_sc as plsc

_HIDDEN = 1024
_LANES = 16
_NCORES = 2
_NSUB = 16
_NW = _NCORES * _NSUB

_CHUNK = 16


def _emb_body(ids_hbm, pids_hbm, word_hbm, ptab_hbm, out_hbm,
              idx_w, idx_p, w0, w1, p0, p1, o0, o1,
              sw0, sw1, sp0, sp1, so0, so1, *, per_w, nchunk, wpb):
    w = (w0, w1)
    p = (p0, p1)
    o = (o0, o1)
    sw = (sw0, sw1)
    sp = (sp0, sp1)
    so = (so0, so1)

    wid = lax.axis_index("s") * _NCORES + lax.axis_index("c")
    bi = wid // wpb
    co = (wid % wpb) * per_w
    pltpu.sync_copy(ids_hbm.at[bi, pl.ds(co, per_w)], idx_w)
    pltpu.sync_copy(pids_hbm.at[bi, pl.ds(co, per_w)], idx_p)

    def gather_pair(ci, b):
        off = ci * _CHUNK
        pltpu.make_async_copy(word_hbm.at[idx_w.at[pl.ds(off, _CHUNK)]],
                              w[b], sw[b]).start()
        pltpu.make_async_copy(ptab_hbm.at[idx_p.at[pl.ds(off, _CHUNK)]],
                              p[b], sp[b]).start()

    def wait_gather(ci, b):
        off = ci * _CHUNK
        pltpu.make_async_copy(word_hbm.at[idx_w.at[pl.ds(off, _CHUNK)]],
                              w[b], sw[b]).wait()
        pltpu.make_async_copy(ptab_hbm.at[idx_p.at[pl.ds(off, _CHUNK)]],
                              p[b], sp[b]).wait()

    def start_out(ci, b):
        pltpu.make_async_copy(
            o[b], out_hbm.at[bi, pl.ds(co + ci * _CHUNK, _CHUNK)],
            so[b]).start()

    def wait_out(ci, b):
        pltpu.make_async_copy(
            o[b], out_hbm.at[bi, pl.ds(co + ci * _CHUNK, _CHUNK)],
            so[b]).wait()

    gather_pair(0, 0)

    def pair_body(g, carry):
        for b in (0, 1):
            ci = 2 * g + b
            if b == 0:
                gather_pair(ci + 1, 1 - b)
            else:
                @pl.when(g < (nchunk // 2) - 1)
                def _():
                    gather_pair(ci + 1, 1 - b)
            wait_gather(ci, b)

            @pl.when(g > 0)
            def _():
                wait_out(ci - 2, b)

            def add_row(r, c2):
                for j in range(_HIDDEN // _LANES):
                    sl = pl.ds(j * _LANES, _LANES)
                    o[b][r, sl] = w[b][r, sl] + p[b][r, sl]
                return c2

            lax.fori_loop(0, _CHUNK, add_row, 0, unroll=False)
            start_out(ci, b)
        return carry

    lax.fori_loop(0, nchunk // 2, pair_body, 0, unroll=False)
    wait_out(nchunk - 2, 0)
    wait_out(nchunk - 1, 1)


def kernel(input_ids, position_ids, word_table, pos_table):
    b, s = input_ids.shape
    n = b * s
    per_w = n // _NW
    nchunk = per_w // _CHUNK
    wpb = s // per_w
    if input_ids.dtype != jnp.int32:
        input_ids = input_ids.astype(jnp.int32)
    if position_ids.dtype != jnp.int32:
        position_ids = position_ids.astype(jnp.int32)


    mesh = plsc.VectorSubcoreMesh(core_axis_name="c", subcore_axis_name="s")
    scratch = [pltpu.VMEM((per_w,), jnp.int32),
               pltpu.VMEM((per_w,), jnp.int32),
               pltpu.VMEM((_CHUNK, _HIDDEN), jnp.float32),
               pltpu.VMEM((_CHUNK, _HIDDEN), jnp.float32),
               pltpu.VMEM((_CHUNK, _HIDDEN), jnp.float32),
               pltpu.VMEM((_CHUNK, _HIDDEN), jnp.float32),
               pltpu.VMEM((_CHUNK, _HIDDEN), jnp.float32),
               pltpu.VMEM((_CHUNK, _HIDDEN), jnp.float32)]
    scratch += [pltpu.SemaphoreType.DMA for _ in range(6)]
    grid_kernel = pl.kernel(
        functools.partial(_emb_body, per_w=per_w, nchunk=nchunk, wpb=wpb),
        mesh=mesh,
        out_type=jax.ShapeDtypeStruct((b, s, _HIDDEN), jnp.float32),
        scratch_types=scratch,
        compiler_params=pltpu.CompilerParams(needs_layout_passes=False),
    )
    return grid_kernel(input_ids, position_ids, word_table, pos_table)
